# baseline (speedup 1.0000x reference)
import functools

import jax
import jax.numpy as jnp
from jax import lax
from jax.experimental import pallas as pl
from jax.experimental.pallas import tpu as pltpu
from jax.experimental.pallas import tpu_sc as plsc

_NC = 2
_NS = 16
_NW = _NC * _NS
_CH = 128
_D = 64


def _sc_gather(pu2, pi2, ug_t, ig_t, um_t, im_t):
    B = pu2.shape[0] * _CH
    bpw = B // _NW
    nch = bpw // _CH
    mesh = plsc.VectorSubcoreMesh(core_axis_name="c", subcore_axis_name="s")

    @functools.partial(
        pl.kernel,
        mesh=mesh,
        out_type=[jax.ShapeDtypeStruct((B, 2 * _D), jnp.float32)] * 4,
        scratch_types=[
            pltpu.VMEM((nch, _CH), jnp.int32),
            pltpu.VMEM((nch, _CH), jnp.int32),
            pltpu.VMEM((_CH, 2 * _D), jnp.float32),
            pltpu.VMEM((_CH, 2 * _D), jnp.float32),
            pltpu.SemaphoreType.DMA,
            pltpu.SemaphoreType.DMA,
            pltpu.SemaphoreType.DMA,
            pltpu.SemaphoreType.DMA,
        ],
    )
    def k(pu_h, pi_h, ug_h, ig_h, um_h, im_h,
          o_ug, o_ig, o_um, o_im,
          gxu, gxi, buf0, buf1, g0, g1, w0, w1):
        wid = lax.axis_index("s") * _NC + lax.axis_index("c")
        base = wid * bpw
        pltpu.sync_copy(pu_h.at[pl.ds(wid * nch, nch)], gxu)
        pltpu.sync_copy(pi_h.at[pl.ds(wid * nch, nch)], gxi)
        specs = ((ug_h, gxu, o_ug), (ig_h, gxi, o_ig),
                 (um_h, gxu, o_um), (im_h, gxi, o_im))
        buf = (buf0, buf1)
        gsem = (g0, g1)
        wsem = (w0, w1)
        ntot = 4 * nch

        def issue(n, b):
            t, c = divmod(n, nch)
            tab, gx, _ = specs[t]
            return pltpu.async_copy(tab.at[gx.at[c]], buf[b], gsem[b])

        pend = [issue(0, 0), None]
        wd = [None, None]
        for n in range(ntot):
            b = n % 2
            if n + 1 < ntot:
                if wd[1 - b] is not None:
                    wd[1 - b].wait()
                    wd[1 - b] = None
                pend[1 - b] = issue(n + 1, 1 - b)
            pend[b].wait()
            t, c = divmod(n, nch)
            out = specs[t][2]
            wd[b] = pltpu.async_copy(
                buf[b], out.at[pl.ds(base + c * _CH, _CH)], wsem[b])
        wd[0].wait()
        wd[1].wait()

    return k(pu2, pi2, ug_t, ig_t, um_t, im_t)


def _mlp_body(ugp, igp, ump, imp, pu, pi, w1u, w1i, b1, w2, b2, w3, b3,
              wg, wh, bf, out):
    pu_m = pu[...] > 0.5
    pi_m = pi[...] > 0.5

    def pick(pairs, m):
        x = pairs[...]
        return jnp.where(m, x[:, _D:], x[:, :_D])

    um = pick(ump, pu_m)
    im = pick(imp, pi_m)
    h = jnp.dot(um, w1u[...], preferred_element_type=jnp.float32)
    h += jnp.dot(im, w1i[...], preferred_element_type=jnp.float32)
    h = jnp.maximum(h + b1[...], 0.0)
    h = jnp.maximum(
        jnp.dot(h, w2[...], preferred_element_type=jnp.float32) + b2[...], 0.0)
    h = jnp.maximum(
        jnp.dot(h, w3[...], preferred_element_type=jnp.float32) + b3[...], 0.0)
    gmf = pick(ugp, pu_m) * pick(igp, pi_m)
    logit = (jnp.dot(gmf, wg[...], preferred_element_type=jnp.float32)
             + jnp.dot(h, wh[...], preferred_element_type=jnp.float32)
             + bf[0, 0])
    out[...] = 1.0 / (1.0 + jnp.exp(-logit))


def kernel(user_ids, item_ids, ue_gmf, ie_gmf, ue_mlp, ie_mlp,
           W1, b1, W2, b2, W3, b3, Wf, bf):
    B = user_ids.shape[0]
    D = ue_gmf.shape[1]
    NU = ue_gmf.shape[0]
    NI = ie_gmf.shape[0]
    ug_t = ue_gmf.reshape(NU // 2, 2 * D)
    ig_t = ie_gmf.reshape(NI // 2, 2 * D)
    um_t = ue_mlp.reshape(NU // 2, 2 * D)
    im_t = ie_mlp.reshape(NI // 2, 2 * D)
    pu2 = (user_ids // 2).reshape(B // _CH, _CH)
    pi2 = (item_ids // 2).reshape(B // _CH, _CH)
    puf = (user_ids % 2).astype(jnp.float32).reshape(B, 1)
    pif = (item_ids % 2).astype(jnp.float32).reshape(B, 1)
    ugp, igp, ump, imp = _sc_gather(pu2, pi2, ug_t, ig_t, um_t, im_t)

    H1 = W1.shape[0]
    H2 = W2.shape[0]
    H3 = W3.shape[0]
    w1u = W1[:, :D].T
    w1i = W1[:, D:].T
    w2t = W2.T
    w3t = W3.T
    wg = Wf[:, :D].T
    wh = Wf[:, D:].T
    b1r = b1.reshape(1, H1)
    b2r = b2.reshape(1, H2)
    b3r = b3.reshape(1, H3)
    bfr = bf.reshape(1, 1)

    bB = 2048
    grid = (B // bB,)
    row_spec = pl.BlockSpec((bB, 2 * D), lambda i: (i, 0))
    par_spec = pl.BlockSpec((bB, 1), lambda i: (i, 0))

    def _w(shape):
        return pl.BlockSpec(shape, lambda i: (0, 0))

    out2 = pl.pallas_call(
        _mlp_body,
        grid=grid,
        in_specs=[
            row_spec, row_spec, row_spec, row_spec,
            par_spec, par_spec,
            _w((D, H1)), _w((D, H1)), _w((1, H1)),
            _w((H1, H2)), _w((1, H2)),
            _w((H2, H3)), _w((1, H3)),
            _w((D, 1)), _w((H3, 1)), _w((1, 1)),
        ],
        out_specs=pl.BlockSpec((bB, 1), lambda i: (i, 0)),
        out_shape=jax.ShapeDtypeStruct((B, 1), jnp.float32),
    )(ugp, igp, ump, imp, puf, pif,
      w1u, w1i, b1r, w2t, b2r, w3t, b3r, wg, wh, bfr)
    return out2.reshape(B)


# speedup vs baseline: 1.1382x; 1.1382x over previous
import functools

import jax
import jax.numpy as jnp
from jax import lax
from jax.experimental import pallas as pl
from jax.experimental.pallas import tpu as pltpu
from jax.experimental.pallas import tpu_sc as plsc

_NC = 2
_NS = 16
_NW = _NC * _NS
_D = 64


def _sc_gather_linear(uid2, iid2, uem, iem):
    ch = 128
    B = uid2.shape[0] * ch
    bpw = B // _NW
    nch = bpw // ch
    mesh = plsc.VectorSubcoreMesh(core_axis_name="c", subcore_axis_name="s")

    @functools.partial(
        pl.kernel,
        mesh=mesh,
        compiler_params=pltpu.CompilerParams(use_tc_tiling_on_sc=False),
        out_type=[jax.ShapeDtypeStruct((B, _D), jnp.float32)] * 2,
        scratch_types=[
            pltpu.VMEM((nch, ch), jnp.int32),
            pltpu.VMEM((nch, ch), jnp.int32),
            pltpu.VMEM((ch, _D), jnp.float32),
            pltpu.VMEM((ch, _D), jnp.float32),
            pltpu.SemaphoreType.DMA,
            pltpu.SemaphoreType.DMA,
            pltpu.SemaphoreType.DMA,
            pltpu.SemaphoreType.DMA,
        ],
    )
    def k(uid_h, iid_h, uem_h, iem_h, o_um, o_im,
          gxu, gxi, buf0, buf1, g0, g1, w0, w1):
        wid = lax.axis_index("s") * _NC + lax.axis_index("c")
        base = wid * bpw
        pltpu.sync_copy(uid_h.at[pl.ds(wid * nch, nch)], gxu)
        pltpu.sync_copy(iid_h.at[pl.ds(wid * nch, nch)], gxi)
        specs = ((uem_h, gxu, o_um), (iem_h, gxi, o_im))
        buf = (buf0, buf1)
        gsem = (g0, g1)
        wsem = (w0, w1)
        ntot = 2 * nch

        def issue(n, b):
            t, c = divmod(n, nch)
            tab, gx, _ = specs[t]
            return pltpu.async_copy(tab.at[gx.at[c]], buf[b], gsem[b])

        pend = [issue(0, 0), None]
        wd = [None, None]
        for n in range(ntot):
            b = n % 2
            if n + 1 < ntot:
                if wd[1 - b] is not None:
                    wd[1 - b].wait()
                    wd[1 - b] = None
                pend[1 - b] = issue(n + 1, 1 - b)
            pend[b].wait()
            t, c = divmod(n, nch)
            out = specs[t][2]
            wd[b] = pltpu.async_copy(
                buf[b], out.at[pl.ds(base + c * ch, ch)], wsem[b])
        wd[0].wait()
        wd[1].wait()

    return k(uid2, iid2, uem, iem)


def _sc_gather_rowdma(urep, irep, ueg, ieg):
    ch = 32
    B = urep.shape[0] * 8
    bpw = B // _NW
    nch = bpw // ch
    mesh = plsc.VectorSubcoreMesh(core_axis_name="c", subcore_axis_name="s")

    @functools.partial(
        pl.kernel,
        mesh=mesh,
        out_type=[jax.ShapeDtypeStruct((B, _D), jnp.float32)] * 2,
        scratch_types=[
            pltpu.VMEM((ch, _D), jnp.float32),
            pltpu.VMEM((ch, _D), jnp.float32),
            pltpu.VMEM((bpw // 8, 128), jnp.int32),
            pltpu.VMEM((bpw // 8, 128), jnp.int32),
            pltpu.SemaphoreType.DMA,
            pltpu.SemaphoreType.DMA,
            pltpu.SemaphoreType.DMA,
            pltpu.SemaphoreType.DMA,
        ],
    )
    def k(uid_h, iid_h, ueg_h, ieg_h, o_ug, o_ig,
          dst0, dst1, idvu, idvi, g0, g1, w0, w1):
        wid = lax.axis_index("s") * _NC + lax.axis_index("c")
        base = wid * bpw
        pltpu.sync_copy(uid_h.at[pl.ds(wid * (bpw // 8), bpw // 8)], idvu)
        pltpu.sync_copy(iid_h.at[pl.ds(wid * (bpw // 8), bpw // 8)], idvi)
        specs = ((ueg_h, idvu, o_ug), (ieg_h, idvi, o_ig))
        dst = (dst0, dst1)
        gsem = (g0, g1)
        wsem = (w0, w1)
        ntot = 2 * nch

        def issue(n, b):
            t, c = divmod(n, nch)
            tab, idv, _ = specs[t]
            dst_b = dst[b]

            @pl.loop(0, ch)
            def _rows(i):
                j = c * ch + i
                v = idv[j // 8, pl.ds((j % 8) * 16, 16)]
                rid = v[0]
                pltpu.make_async_copy(
                    tab.at[pl.ds(rid, 1)], dst_b.at[pl.ds(i, 1)], gsem[b]
                ).start()

        def drain(n, b):
            tab = specs[divmod(n, nch)[0]][0]
            pltpu.make_async_copy(tab.at[pl.ds(0, ch)], dst[b], gsem[b]).wait()

        wd = [None, None]
        issue(0, 0)
        for n in range(ntot):
            b = n % 2
            if n + 1 < ntot:
                if wd[1 - b] is not None:
                    wd[1 - b].wait()
                    wd[1 - b] = None
                issue(n + 1, 1 - b)
            drain(n, b)
            t, c = divmod(n, nch)
            out = specs[t][2]
            wd[b] = pltpu.async_copy(
                dst[b], out.at[pl.ds(base + c * ch, ch)], wsem[b])
        wd[0].wait()
        wd[1].wait()

    return k(urep, irep, ueg, ieg)


def _mlp_body(ug, ig, um, im, w1u, w1i, b1, w2, b2, w3, b3, wg, wh, bf, out):
    h = jnp.dot(um[...], w1u[...], preferred_element_type=jnp.float32)
    h += jnp.dot(im[...], w1i[...], preferred_element_type=jnp.float32)
    h = jnp.maximum(h + b1[...], 0.0)
    h = jnp.maximum(
        jnp.dot(h, w2[...], preferred_element_type=jnp.float32) + b2[...], 0.0)
    h = jnp.maximum(
        jnp.dot(h, w3[...], preferred_element_type=jnp.float32) + b3[...], 0.0)
    gmf = ug[...] * ig[...]
    logit = (jnp.dot(gmf, wg[...], preferred_element_type=jnp.float32)
             + jnp.dot(h, wh[...], preferred_element_type=jnp.float32)
             + bf[0, 0])
    out[...] = 1.0 / (1.0 + jnp.exp(-logit))


def kernel(user_ids, item_ids, ue_gmf, ie_gmf, ue_mlp, ie_mlp,
           W1, b1, W2, b2, W3, b3, Wf, bf):
    B = user_ids.shape[0]
    D = ue_gmf.shape[1]
    uid2 = user_ids.reshape(B // 128, 128)
    iid2 = item_ids.reshape(B // 128, 128)
    urep = jnp.broadcast_to(user_ids[:, None], (B, 16)).reshape(B // 8, 128)
    irep = jnp.broadcast_to(item_ids[:, None], (B, 16)).reshape(B // 8, 128)
    um, im = _sc_gather_linear(uid2, iid2, ue_mlp, ie_mlp)
    ug, ig = _sc_gather_rowdma(urep, irep, ue_gmf, ie_gmf)

    H1 = W1.shape[0]
    H2 = W2.shape[0]
    H3 = W3.shape[0]
    w1u = W1[:, :D].T
    w1i = W1[:, D:].T
    w2t = W2.T
    w3t = W3.T
    wg = Wf[:, :D].T
    wh = Wf[:, D:].T
    b1r = b1.reshape(1, H1)
    b2r = b2.reshape(1, H2)
    b3r = b3.reshape(1, H3)
    bfr = bf.reshape(1, 1)

    bB = 2048
    grid = (B // bB,)
    row_spec = pl.BlockSpec((bB, D), lambda i: (i, 0))

    def _w(shape):
        return pl.BlockSpec(shape, lambda i: (0, 0))

    out2 = pl.pallas_call(
        _mlp_body,
        grid=grid,
        in_specs=[
            row_spec, row_spec, row_spec, row_spec,
            _w((D, H1)), _w((D, H1)), _w((1, H1)),
            _w((H1, H2)), _w((1, H2)),
            _w((H2, H3)), _w((1, H3)),
            _w((D, 1)), _w((H3, 1)), _w((1, 1)),
        ],
        out_specs=pl.BlockSpec((bB, 1), lambda i: (i, 0)),
        out_shape=jax.ShapeDtypeStruct((B, 1), jnp.float32),
    )(ug, ig, um, im, w1u, w1i, b1r, w2t, b2r, w3t, b3r, wg, wh, bfr)
    return out2.reshape(B)


# speedup vs baseline: 1.4751x; 1.2960x over previous
import functools

import jax
import jax.numpy as jnp
from jax import lax
from jax.experimental import pallas as pl
from jax.experimental.pallas import tpu as pltpu
from jax.experimental.pallas import tpu_sc as plsc

_NC = 2
_NS = 16
_NW = _NC * _NS
_CH = 32
_D = 64


def _sc_gather(urep, irep, ueg, ieg, uem, iem):
    B = urep.shape[0] * 8
    bpw = B // _NW
    nch = bpw // _CH
    mesh = plsc.VectorSubcoreMesh(core_axis_name="c", subcore_axis_name="s")

    @functools.partial(
        pl.kernel,
        mesh=mesh,
        out_type=[jax.ShapeDtypeStruct((B, _D), jnp.float32)] * 4,
        scratch_types=[
            pltpu.VMEM((_CH, _D), jnp.float32),
            pltpu.VMEM((_CH, _D), jnp.float32),
            pltpu.VMEM((bpw // 8, 128), jnp.int32),
            pltpu.VMEM((bpw // 8, 128), jnp.int32),
            pltpu.SemaphoreType.DMA,
            pltpu.SemaphoreType.DMA,
            pltpu.SemaphoreType.DMA,
            pltpu.SemaphoreType.DMA,
        ],
    )
    def k(uid_h, iid_h, ueg_h, ieg_h, uem_h, iem_h,
          o_ug, o_ig, o_um, o_im,
          dst0, dst1, idvu, idvi, g0, g1, w0, w1):
        wid = lax.axis_index("s") * _NC + lax.axis_index("c")
        base = wid * bpw
        pltpu.sync_copy(uid_h.at[pl.ds(wid * (bpw // 8), bpw // 8)], idvu)
        pltpu.sync_copy(iid_h.at[pl.ds(wid * (bpw // 8), bpw // 8)], idvi)
        specs = ((ueg_h, idvu, o_ug), (ieg_h, idvi, o_ig),
                 (uem_h, idvu, o_um), (iem_h, idvi, o_im))
        dst = (dst0, dst1)
        gsem = (g0, g1)
        wsem = (w0, w1)
        ntot = 4 * nch

        def issue(n, b):
            t, c = divmod(n, nch)
            tab, idv, _ = specs[t]
            dst_b = dst[b]

            @pl.loop(0, _CH)
            def _rows(i):
                j = c * _CH + i
                v = idv[j // 8, pl.ds((j % 8) * 16, 16)]
                rid = v[0]
                pltpu.make_async_copy(
                    tab.at[pl.ds(rid, 1)], dst_b.at[pl.ds(i, 1)], gsem[b]
                ).start()

        def drain(n, b):
            tab = specs[divmod(n, nch)[0]][0]
            pltpu.make_async_copy(tab.at[pl.ds(0, _CH)], dst[b], gsem[b]).wait()

        wd = [None, None]
        issue(0, 0)
        for n in range(ntot):
            b = n % 2
            if n + 1 < ntot:
                if wd[1 - b] is not None:
                    wd[1 - b].wait()
                    wd[1 - b] = None
                issue(n + 1, 1 - b)
            drain(n, b)
            t, c = divmod(n, nch)
            out = specs[t][2]
            wd[b] = pltpu.async_copy(
                dst[b], out.at[pl.ds(base + c * _CH, _CH)], wsem[b])
        wd[0].wait()
        wd[1].wait()

    return k(urep, irep, ueg, ieg, uem, iem)


def _mlp_body(ug, ig, um, im, w1u, w1i, b1, w2, b2, w3, b3, wg, wh, bf, out):
    h = jnp.dot(um[...], w1u[...], preferred_element_type=jnp.float32)
    h += jnp.dot(im[...], w1i[...], preferred_element_type=jnp.float32)
    h = jnp.maximum(h + b1[...], 0.0)
    h = jnp.maximum(
        jnp.dot(h, w2[...], preferred_element_type=jnp.float32) + b2[...], 0.0)
    h = jnp.maximum(
        jnp.dot(h, w3[...], preferred_element_type=jnp.float32) + b3[...], 0.0)
    gmf = ug[...] * ig[...]
    logit = (jnp.dot(gmf, wg[...], preferred_element_type=jnp.float32)
             + jnp.dot(h, wh[...], preferred_element_type=jnp.float32)
             + bf[0, 0])
    out[...] = 1.0 / (1.0 + jnp.exp(-logit))


def kernel(user_ids, item_ids, ue_gmf, ie_gmf, ue_mlp, ie_mlp,
           W1, b1, W2, b2, W3, b3, Wf, bf):
    B = user_ids.shape[0]
    D = ue_gmf.shape[1]
    urep = jnp.broadcast_to(user_ids[:, None], (B, 16)).reshape(B // 8, 128)
    irep = jnp.broadcast_to(item_ids[:, None], (B, 16)).reshape(B // 8, 128)
    ug, ig, um, im = _sc_gather(urep, irep,
                                ue_gmf, ie_gmf, ue_mlp, ie_mlp)

    H1 = W1.shape[0]
    H2 = W2.shape[0]
    H3 = W3.shape[0]
    w1u = W1[:, :D].T
    w1i = W1[:, D:].T
    w2t = W2.T
    w3t = W3.T
    wg = Wf[:, :D].T
    wh = Wf[:, D:].T
    b1r = b1.reshape(1, H1)
    b2r = b2.reshape(1, H2)
    b3r = b3.reshape(1, H3)
    bfr = bf.reshape(1, 1)

    bB = 2048
    grid = (B // bB,)
    row_spec = pl.BlockSpec((bB, D), lambda i: (i, 0))

    def _w(shape):
        return pl.BlockSpec(shape, lambda i: (0, 0))

    out2 = pl.pallas_call(
        _mlp_body,
        grid=grid,
        in_specs=[
            row_spec, row_spec, row_spec, row_spec,
            _w((D, H1)), _w((D, H1)), _w((1, H1)),
            _w((H1, H2)), _w((1, H2)),
            _w((H2, H3)), _w((1, H3)),
            _w((D, 1)), _w((H3, 1)), _w((1, 1)),
        ],
        out_specs=pl.BlockSpec((bB, 1), lambda i: (i, 0)),
        out_shape=jax.ShapeDtypeStruct((B, 1), jnp.float32),
    )(ug, ig, um, im, w1u, w1i, b1r, w2t, b2r, w3t, b3r, wg, wh, bfr)
    return out2.reshape(B)
